# baseline (speedup 1.0000x reference)
import functools

import jax
import jax.numpy as jnp
from jax import lax
from jax.experimental import pallas as pl
from jax.experimental.pallas import tpu as pltpu, tpu_sc as plsc

NC, NS = 2, 16
NW = NC * NS
NB = 4
T_TOT = 8192
D_EMB = 64
ND1 = D_EMB // 8
NT1 = T_TOT // 128
TBLK = NW // NB
B_PER_W = T_TOT // TBLK
N_CHUNKS = B_PER_W // 128

_mesh = plsc.VectorSubcoreMesh(core_axis_name="c", subcore_axis_name="s")


@functools.partial(
    pl.kernel,
    mesh=_mesh,
    out_type=jax.ShapeDtypeStruct((NB, ND1, NT1, 8, 128), jnp.float32),
    scratch_types=[
        pltpu.VMEM((N_CHUNKS, 128), jnp.int32),
        pltpu.VMEM((B_PER_W, D_EMB), jnp.float32),
        pltpu.VMEM((2, ND1, 8, 128), jnp.float32),
        pltpu.SemaphoreType.DMA,
        [pltpu.SemaphoreType.DMA] * N_CHUNKS,
        [pltpu.SemaphoreType.DMA] * 2,
    ],
    compiler_params=pltpu.CompilerParams(
        use_tc_tiling_on_sc=False, needs_layout_passes=False
    ),
)
def _gather(x_ph, tbl_hbm, out_ph, idx_v, rows_v, bufs, isem, gsems, ssems):
    wid = lax.axis_index("s") * NC + lax.axis_index("c")
    b = wid // TBLK
    tblk = wid % TBLK
    pltpu.async_copy(x_ph.at[pl.ds(tblk * N_CHUNKS, N_CHUNKS), b], idx_v, isem).wait()
    gh = []
    for j in range(N_CHUNKS):
        gh.append(
            pltpu.async_copy(
                tbl_hbm.at[idx_v.at[j]],
                rows_v.at[pl.ds(j * 128, 128)],
                gsems[j],
            )
        )
    iot = lax.iota(jnp.int32, 16)
    cds = [jnp.full((16,), d, jnp.int32) for d in range(D_EMB)]
    store_h = [None, None]
    for j in range(N_CHUNKS):
        slot = j % 2
        gh[j].wait()
        if store_h[slot] is not None:
            store_h[slot].wait()
        for d1 in range(ND1):

            def tile(kt, _, j=j, d1=d1, slot=slot):
                idx_r = iot + (j * 128 + kt * 16)
                for d0 in range(8):
                    v = plsc.load_gather(rows_v, [idx_r, cds[d1 * 8 + d0]])
                    bufs[slot, d1, d0, pl.ds(kt * 16, 16)] = v
                return 0

            lax.fori_loop(0, 8, tile, 0)
        store_h[slot] = pltpu.async_copy(
            bufs.at[slot],
            out_ph.at[b, :, tblk * N_CHUNKS + j],
            ssems[slot],
        )
    store_h[0].wait()
    store_h[1].wait()


def kernel(x, table):
    x_ph = x.reshape(NB, NT1, 128).transpose(1, 0, 2)
    res = _gather(x_ph, table)
    return res.transpose(0, 2, 4, 1, 3).reshape(NB, T_TOT, D_EMB)


# speedup vs baseline: 1.2341x; 1.2341x over previous
import functools

import jax
import jax.numpy as jnp
from jax import lax
from jax.experimental import pallas as pl
from jax.experimental.pallas import tpu as pltpu, tpu_sc as plsc

NC, NS = 2, 16
NW = NC * NS
NB = 4
T_TOT = 8192
D_EMB = 64
ND1 = D_EMB // 8
NT1 = T_TOT // 128
TBLK = NW // NB
B_PER_W = T_TOT // TBLK
N_CHUNKS = B_PER_W // 128

_mesh = plsc.VectorSubcoreMesh(core_axis_name="c", subcore_axis_name="s")


def _transpose_tile(j, d1, slot, iot, cds, rows_v, bufs):
    @plsc.parallel_loop(0, 8, unroll=2)
    def tile(kt):
        idx_r = iot + (j * 128 + kt * 16)
        for d0 in range(8):
            v = plsc.load_gather(rows_v, [idx_r, cds[d1 * 8 + d0]])
            bufs[slot, d1, d0, pl.ds(kt * 16, 16)] = v


@functools.partial(
    pl.kernel,
    mesh=_mesh,
    out_type=jax.ShapeDtypeStruct((NB, ND1, NT1, 8, 128), jnp.float32),
    scratch_types=[
        pltpu.VMEM((N_CHUNKS, 128), jnp.int32),
        pltpu.VMEM((B_PER_W, D_EMB), jnp.float32),
        pltpu.VMEM((2, ND1, 8, 128), jnp.float32),
        pltpu.SemaphoreType.DMA,
        [pltpu.SemaphoreType.DMA] * N_CHUNKS,
        [pltpu.SemaphoreType.DMA] * 2,
    ],
    compiler_params=pltpu.CompilerParams(
        use_tc_tiling_on_sc=False, needs_layout_passes=False
    ),
)
def _gather(x_ph, tbl_hbm, out_ph, idx_v, rows_v, bufs, isem, gsems, ssems):
    wid = lax.axis_index("s") * NC + lax.axis_index("c")
    b = wid // TBLK
    tblk = wid % TBLK
    pltpu.async_copy(x_ph.at[pl.ds(tblk * N_CHUNKS, N_CHUNKS), b], idx_v, isem).wait()
    gh = []
    for j in range(N_CHUNKS):
        gh.append(
            pltpu.async_copy(
                tbl_hbm.at[idx_v.at[j]],
                rows_v.at[pl.ds(j * 128, 128)],
                gsems[j],
            )
        )
    iot = lax.iota(jnp.int32, 16)
    cds = [jnp.full((16,), d, jnp.int32) for d in range(D_EMB)]
    store_h = [None, None]
    for j in range(N_CHUNKS):
        slot = j % 2
        gh[j].wait()
        if store_h[slot] is not None:
            store_h[slot].wait()
        for d1 in range(ND1):
            _transpose_tile(j, d1, slot, iot, cds, rows_v, bufs)
        store_h[slot] = pltpu.async_copy(
            bufs.at[slot],
            out_ph.at[b, :, tblk * N_CHUNKS + j],
            ssems[slot],
        )
    store_h[0].wait()
    store_h[1].wait()


def kernel(x, table):
    x_ph = x.reshape(NB, NT1, 128).transpose(1, 0, 2)
    res = _gather(x_ph, table)
    return res.transpose(0, 2, 4, 1, 3).reshape(NB, T_TOT, D_EMB)


# speedup vs baseline: 2.7478x; 2.2265x over previous
import jax
import jax.numpy as jnp
from jax import lax
from jax.experimental import pallas as pl
from jax.experimental.pallas import tpu as pltpu, tpu_sc as plsc

NC, NS = 2, 16
NW = NC * NS
NB = 4
T_TOT = 8192
D_EMB = 64
ND1 = D_EMB // 8
NT1 = T_TOT // 128
TBLK = NW // NB
B_PER_W = T_TOT // TBLK
N_CHUNKS = B_PER_W // 128
PITCH = 129

_mesh = plsc.VectorSubcoreMesh(core_axis_name="c", subcore_axis_name="s")


def _transpose_chunk(j, rows_v, buf, d1s, d0s):
    @plsc.parallel_loop(0, 128, unroll=2)
    def row(tl):
        tv = jnp.full((16,), tl, jnp.int32)
        for dg in range(D_EMB // 16):
            v = rows_v[j * 128 + tl, pl.ds(dg * 16, 16)]
            plsc.store_scatter(buf, [d1s[dg], d0s[dg], tv], v)


@pl.kernel(
    mesh=_mesh,
    out_type=jax.ShapeDtypeStruct((NB, ND1, NT1, 8, 128), jnp.float32),
    scratch_types=[
        pltpu.VMEM((N_CHUNKS, 128), jnp.int32),
        pltpu.VMEM((B_PER_W, D_EMB), jnp.float32),
        pltpu.VMEM((ND1, 8, PITCH), jnp.float32),
        pltpu.VMEM((ND1, 8, PITCH), jnp.float32),
        pltpu.SemaphoreType.DMA,
        [pltpu.SemaphoreType.DMA] * N_CHUNKS,
        [pltpu.SemaphoreType.DMA] * 2,
    ],
    compiler_params=pltpu.CompilerParams(
        use_tc_tiling_on_sc=False, needs_layout_passes=False
    ),
)
def _gather(x_ph, tbl_hbm, out_ph, idx_v, rows_v, buf_a, buf_b, isem, gsems, ssems):
    wid = lax.axis_index("s") * NC + lax.axis_index("c")
    b = wid // TBLK
    tblk = wid % TBLK
    pltpu.async_copy(x_ph.at[pl.ds(tblk * N_CHUNKS, N_CHUNKS), b], idx_v, isem).wait()
    gh = []
    for j in range(N_CHUNKS):
        gh.append(
            pltpu.async_copy(
                tbl_hbm.at[idx_v.at[j]],
                rows_v.at[pl.ds(j * 128, 128)],
                gsems[j],
            )
        )
    iot = lax.iota(jnp.int32, 16)
    d1s = [(iot + dg * 16) >> 3 for dg in range(D_EMB // 16)]
    d0s = [(iot + dg * 16) & 7 for dg in range(D_EMB // 16)]
    bufs = [buf_a, buf_b]
    store_h = [None, None]
    for j in range(N_CHUNKS):
        slot = j % 2
        gh[j].wait()
        if store_h[slot] is not None:
            store_h[slot].wait()
        _transpose_chunk(j, rows_v, bufs[slot], d1s, d0s)
        store_h[slot] = pltpu.async_copy(
            bufs[slot].at[:, :, pl.ds(0, 128)],
            out_ph.at[b, :, tblk * N_CHUNKS + j],
            ssems[slot],
        )
    store_h[0].wait()
    store_h[1].wait()


def kernel(x, table):
    x_ph = x.reshape(NB, NT1, 128).transpose(1, 0, 2)
    res = _gather(x_ph, table)
    return res.transpose(0, 2, 4, 1, 3).reshape(NB, T_TOT, D_EMB)
